# Initial kernel scaffold; baseline (speedup 1.0000x reference)
import jax
import jax.numpy as jnp
from jax.experimental import pallas as pl


def kernel(hidden_states, W_router, Wg_dyn, Wu_dyn, Wd_dyn, Wg_sh, Wu_sh, Wd_sh):
    raise NotImplementedError("write your pallas kernel here")


# speedup vs baseline: 1.3015x; 1.3015x over previous
import functools

import jax
import jax.numpy as jnp
from jax.experimental import pallas as pl
from jax.experimental.pallas import tpu as pltpu

E_DYN = 8
E_FIX = 1
NE = E_DYN + E_FIX
TOP_K = 2
D = 2048
DFF = 512
EPS2 = 0.02
TB = 256
NT = 2048 // TB

NEG_INF = float("-inf")


def _silu(x):
    return x * jax.nn.sigmoid(x)


def _routing_weights(logits):
    scores = logits[:, :E_DYN]
    io8 = jax.lax.broadcasted_iota(jnp.int32, scores.shape, 1)

    thr1 = jnp.max(scores, axis=1, keepdims=True)
    a1 = jnp.min(jnp.where(scores == thr1, io8, E_DYN), axis=1, keepdims=True)
    factor1 = jnp.maximum(jnp.abs(scores), jnp.abs(thr1))
    m1 = (thr1 - scores) / factor1 > EPS2
    mg1 = jnp.where(m1, NEG_INF, scores)
    g1 = jax.nn.softmax(mg1, axis=-1)
    mult1 = jnp.sum(jnp.where(io8 == a1, g1, 0.0), axis=1, keepdims=True)

    masked2 = jnp.where(io8 == a1, NEG_INF, scores)
    thr2 = jnp.max(masked2, axis=1, keepdims=True)
    a2 = jnp.min(jnp.where(masked2 == thr2, io8, E_DYN), axis=1, keepdims=True)
    factor2 = jnp.maximum(jnp.abs(scores), jnp.abs(thr2))
    m2 = (thr2 - scores) / factor2 > EPS2
    mg2 = jnp.where(m2, NEG_INF, masked2)
    g2 = jax.nn.softmax(mg2, axis=-1)
    mult2 = jnp.sum(jnp.where(io8 == a2, g2, 0.0), axis=1, keepdims=True)

    io9 = jax.lax.broadcasted_iota(jnp.int32, logits.shape, 1)
    sel = (io9 == a1) | (io9 == a2) | (io9 == E_DYN)
    gw_logits = jnp.where(sel, logits, NEG_INF)
    gw = jax.nn.softmax(gw_logits, axis=-1)
    sum_gdyn = jnp.sum(gw[:, :E_DYN], axis=1, keepdims=True)
    g_fix = gw[:, E_DYN:]

    w1 = mult1 * sum_gdyn
    w2 = mult2 * sum_gdyn
    wdyn = jnp.where(io8 == a1, w1, 0.0) + jnp.where(io8 == a2, w2, 0.0)
    return jnp.concatenate([wdyn, g_fix], axis=1)


def _dense_body(x_ref, wr_ref, wg_d_ref, wu_d_ref, wd_d_ref,
                wg_s_ref, wu_s_ref, wd_s_ref, out_ref, w_scr):
    e = pl.program_id(1)
    x = x_ref[...]

    @pl.when(e == 0)
    def _():
        logits = jnp.dot(x, wr_ref[...], preferred_element_type=jnp.float32)
        w_scr[...] = _routing_weights(logits)

    lane = jax.lax.broadcasted_iota(jnp.int32, (TB, NE), 1)
    weight = jnp.sum(jnp.where(lane == e, w_scr[...], 0.0), axis=1,
                     keepdims=True)

    @pl.when(e < E_DYN)
    def _():
        h = _silu(jnp.dot(x, wg_d_ref[0], preferred_element_type=jnp.float32))
        h = h * jnp.dot(x, wu_d_ref[0], preferred_element_type=jnp.float32)
        y = weight * jnp.dot(h, wd_d_ref[0], preferred_element_type=jnp.float32)

        @pl.when(e == 0)
        def _():
            out_ref[...] = y

        @pl.when(e > 0)
        def _():
            out_ref[...] += y

    @pl.when(e == E_DYN)
    def _():
        h = _silu(jnp.dot(x, wg_s_ref[0], preferred_element_type=jnp.float32))
        h = h * jnp.dot(x, wu_s_ref[0], preferred_element_type=jnp.float32)
        out_ref[...] += weight * jnp.dot(h, wd_s_ref[0],
                                         preferred_element_type=jnp.float32)


@jax.jit
def _moe_dense(x, W_router, Wg_dyn, Wu_dyn, Wd_dyn, Wg_sh, Wu_sh, Wd_sh):
    T = x.shape[0]
    clamp7 = lambda t, e: (jnp.minimum(e, 7), 0, 0)
    grid_spec = pltpu.PrefetchScalarGridSpec(
        num_scalar_prefetch=0,
        grid=(NT, NE),
        in_specs=[
            pl.BlockSpec((TB, D), lambda t, e: (t, 0)),
            pl.BlockSpec((D, NE), lambda t, e: (0, 0)),
            pl.BlockSpec((1, D, DFF), clamp7),
            pl.BlockSpec((1, D, DFF), clamp7),
            pl.BlockSpec((1, DFF, D), clamp7),
            pl.BlockSpec((1, D, DFF), lambda t, e: (0, 0, 0)),
            pl.BlockSpec((1, D, DFF), lambda t, e: (0, 0, 0)),
            pl.BlockSpec((1, DFF, D), lambda t, e: (0, 0, 0)),
        ],
        out_specs=pl.BlockSpec((TB, D), lambda t, e: (t, 0)),
        scratch_shapes=[pltpu.VMEM((TB, NE), jnp.float32)],
    )
    return pl.pallas_call(
        _dense_body,
        grid_spec=grid_spec,
        out_shape=jax.ShapeDtypeStruct((T, D), jnp.float32),
        compiler_params=pltpu.CompilerParams(
            dimension_semantics=("arbitrary", "arbitrary"),
        ),
    )(x, W_router, Wg_dyn, Wu_dyn, Wd_dyn, Wg_sh, Wu_sh, Wd_sh)


def kernel(hidden_states, W_router, Wg_dyn, Wu_dyn, Wd_dyn, Wg_sh, Wu_sh, Wd_sh):
    B, S, Dm = hidden_states.shape
    x = hidden_states.reshape(-1, Dm)
    out = _moe_dense(x, W_router, Wg_dyn, Wu_dyn, Wd_dyn, Wg_sh, Wu_sh, Wd_sh)
    return out.reshape(B, S, Dm)


# speedup vs baseline: 1.7290x; 1.3285x over previous
import functools

import jax
import jax.numpy as jnp
from jax.experimental import pallas as pl
from jax.experimental.pallas import tpu as pltpu

E_DYN = 8
E_FIX = 1
NE = E_DYN + E_FIX
TOP_K = 2
D = 2048
DFF = 512
EPS2 = 0.02
TB = 512
NT = 2048 // TB

NEG_INF = float("-inf")


def _silu(x):
    return x * jax.nn.sigmoid(x)


def _routing_weights(logits):
    scores = logits[:, :E_DYN]
    io8 = jax.lax.broadcasted_iota(jnp.int32, scores.shape, 1)

    thr1 = jnp.max(scores, axis=1, keepdims=True)
    a1 = jnp.min(jnp.where(scores == thr1, io8, E_DYN), axis=1, keepdims=True)
    factor1 = jnp.maximum(jnp.abs(scores), jnp.abs(thr1))
    m1 = (thr1 - scores) / factor1 > EPS2
    mg1 = jnp.where(m1, NEG_INF, scores)
    g1 = jax.nn.softmax(mg1, axis=-1)
    mult1 = jnp.sum(jnp.where(io8 == a1, g1, 0.0), axis=1, keepdims=True)

    masked2 = jnp.where(io8 == a1, NEG_INF, scores)
    thr2 = jnp.max(masked2, axis=1, keepdims=True)
    a2 = jnp.min(jnp.where(masked2 == thr2, io8, E_DYN), axis=1, keepdims=True)
    factor2 = jnp.maximum(jnp.abs(scores), jnp.abs(thr2))
    m2 = (thr2 - scores) / factor2 > EPS2
    mg2 = jnp.where(m2, NEG_INF, masked2)
    g2 = jax.nn.softmax(mg2, axis=-1)
    mult2 = jnp.sum(jnp.where(io8 == a2, g2, 0.0), axis=1, keepdims=True)

    io9 = jax.lax.broadcasted_iota(jnp.int32, logits.shape, 1)
    sel = (io9 == a1) | (io9 == a2) | (io9 == E_DYN)
    gw_logits = jnp.where(sel, logits, NEG_INF)
    gw = jax.nn.softmax(gw_logits, axis=-1)
    sum_gdyn = jnp.sum(gw[:, :E_DYN], axis=1, keepdims=True)
    g_fix = gw[:, E_DYN:]

    w1 = mult1 * sum_gdyn
    w2 = mult2 * sum_gdyn
    wdyn = jnp.where(io8 == a1, w1, 0.0) + jnp.where(io8 == a2, w2, 0.0)
    return jnp.concatenate([wdyn, g_fix], axis=1)


def _dense_body(x_ref, wr_ref, wg_d_ref, wu_d_ref, wd_d_ref,
                wg_s_ref, wu_s_ref, wd_s_ref, out_ref, w_scr, xb_scr):
    e = pl.program_id(1)

    @pl.when(e == 0)
    def _():
        x = x_ref[...]
        logits = jnp.dot(x, wr_ref[...], preferred_element_type=jnp.float32)
        w_scr[...] = _routing_weights(logits)
        xb_scr[...] = x.astype(jnp.bfloat16)

    xb = xb_scr[...]
    lane = jax.lax.broadcasted_iota(jnp.int32, (TB, NE), 1)
    weight = jnp.sum(jnp.where(lane == e, w_scr[...], 0.0), axis=1,
                     keepdims=True)

    @pl.when(e < E_DYN)
    def _():
        h = _silu(jnp.dot(xb, wg_d_ref[0], preferred_element_type=jnp.float32))
        h = h * jnp.dot(xb, wu_d_ref[0], preferred_element_type=jnp.float32)
        y = weight * jnp.dot(h.astype(jnp.bfloat16), wd_d_ref[0],
                             preferred_element_type=jnp.float32)

        @pl.when(e == 0)
        def _():
            out_ref[...] = y

        @pl.when(e > 0)
        def _():
            out_ref[...] += y

    @pl.when(e == E_DYN)
    def _():
        h = _silu(jnp.dot(xb, wg_s_ref[0], preferred_element_type=jnp.float32))
        h = h * jnp.dot(xb, wu_s_ref[0], preferred_element_type=jnp.float32)
        out_ref[...] += weight * jnp.dot(h.astype(jnp.bfloat16), wd_s_ref[0],
                                         preferred_element_type=jnp.float32)


@jax.jit
def _moe_dense(x, W_router, Wg_dyn, Wu_dyn, Wd_dyn, Wg_sh, Wu_sh, Wd_sh):
    T = x.shape[0]
    clamp7 = lambda t, e: (jnp.minimum(e, 7), 0, 0)
    grid_spec = pltpu.PrefetchScalarGridSpec(
        num_scalar_prefetch=0,
        grid=(NT, NE),
        in_specs=[
            pl.BlockSpec((TB, D), lambda t, e: (t, 0)),
            pl.BlockSpec((D, NE), lambda t, e: (0, 0)),
            pl.BlockSpec((1, D, DFF), clamp7),
            pl.BlockSpec((1, D, DFF), clamp7),
            pl.BlockSpec((1, DFF, D), clamp7),
            pl.BlockSpec((1, D, DFF), lambda t, e: (0, 0, 0)),
            pl.BlockSpec((1, D, DFF), lambda t, e: (0, 0, 0)),
            pl.BlockSpec((1, DFF, D), lambda t, e: (0, 0, 0)),
        ],
        out_specs=pl.BlockSpec((TB, D), lambda t, e: (t, 0)),
        scratch_shapes=[pltpu.VMEM((TB, NE), jnp.float32),
                        pltpu.VMEM((TB, D), jnp.bfloat16)],
    )
    return pl.pallas_call(
        _dense_body,
        grid_spec=grid_spec,
        out_shape=jax.ShapeDtypeStruct((T, D), jnp.float32),
        compiler_params=pltpu.CompilerParams(
            dimension_semantics=("arbitrary", "arbitrary"),
        ),
    )(x, W_router, Wg_dyn, Wu_dyn, Wd_dyn, Wg_sh, Wu_sh, Wd_sh)


def kernel(hidden_states, W_router, Wg_dyn, Wu_dyn, Wd_dyn, Wg_sh, Wu_sh, Wd_sh):
    B, S, Dm = hidden_states.shape
    x = hidden_states.reshape(-1, Dm)
    bf = jnp.bfloat16
    out = _moe_dense(x, W_router,
                     Wg_dyn.astype(bf), Wu_dyn.astype(bf), Wd_dyn.astype(bf),
                     Wg_sh.astype(bf), Wu_sh.astype(bf), Wd_sh.astype(bf))
    return out.reshape(B, S, Dm)
